# Initial kernel scaffold; baseline (speedup 1.0000x reference)
import jax
import jax.numpy as jnp
from jax.experimental import pallas as pl


def kernel(x, edge_index, W1, b1, W2, b2):
    raise NotImplementedError("write your pallas kernel here")


# speedup vs baseline: 7.0984x; 7.0984x over previous
import functools

import jax
import jax.numpy as jnp
from jax import lax
from jax.experimental import pallas as pl
from jax.experimental.pallas import tpu as pltpu
from jax.experimental.pallas import tpu_sc as plsc

NC = 2
NS = 16
CHUNK = 128


def _mesh():
    return plsc.VectorSubcoreMesh(
        core_axis_name="c", subcore_axis_name="s", num_cores=NC,
        num_subcores=NS)


def _ranged(fn, s, rows_pt, n):
    last = n - (NS - 1) * rows_pt
    pl.when(s != NS - 1)(functools.partial(fn, rows_pt))
    pl.when(s == NS - 1)(functools.partial(fn, last))


def _make_deg_kernel(n, np_rows, e_pad):
    n_chunks = e_pad // (NC * NS * CHUNK)
    rows_pt = np_rows // NS

    @functools.partial(
        pl.kernel,
        out_type=jax.ShapeDtypeStruct((NC, n, 128), jnp.float32),
        mesh=_mesh(),
        scratch_types=[
            pltpu.VMEM((n_chunks, CHUNK), jnp.int32),
            pltpu.VMEM((CHUNK, 128), jnp.float32),
            pltpu.VMEM_SHARED((np_rows, 128), jnp.float32),
        ],
    )
    def deg_k(dst3d, ones_hbm, zeros_hbm, out_hbm, didx, ones_v, acc):
        c = lax.axis_index("c")
        s = lax.axis_index("s")
        wid = s * NC + c
        r0 = s * rows_pt
        pltpu.sync_copy(dst3d.at[wid], didx)
        pltpu.sync_copy(ones_hbm, ones_v)

        def init(nr):
            pltpu.sync_copy(zeros_hbm.at[pl.ds(r0, nr), :],
                            acc.at[pl.ds(r0, nr), :])
        _ranged(init, s, rows_pt, n)
        plsc.subcore_barrier()

        def body(j, carry):
            pltpu.sync_copy(ones_v, acc.at[didx.at[j]], add=True)
            return carry
        lax.fori_loop(0, n_chunks, body, 0)
        plsc.subcore_barrier()

        def emit(nr):
            pltpu.sync_copy(acc.at[pl.ds(r0, nr), :],
                            out_hbm.at[c, pl.ds(r0, nr), :])
        _ranged(emit, s, rows_pt, n)

    return deg_k


def _make_scatter1(n, np_rows, e_pad, dh):
    n_chunks = e_pad // (NS * CHUNK)
    rows_pt = np_rows // NS

    @functools.partial(
        pl.kernel,
        out_type=jax.ShapeDtypeStruct((NC, n, dh), jnp.float32),
        mesh=_mesh(),
        scratch_types=[
            pltpu.VMEM((n_chunks, CHUNK), jnp.int32),
            pltpu.VMEM((n_chunks, CHUNK), jnp.int32),
            pltpu.VMEM((CHUNK, dh), jnp.float32),
            pltpu.VMEM_SHARED((np_rows, dh), jnp.float32),
            pltpu.SemaphoreType.DMA,
        ],
    )
    def sc1_k(src3d, dst3d, hs0, hs1, out_hbm, sidx, didx, rows_v, acc, sem):
        c = lax.axis_index("c")
        s = lax.axis_index("s")
        r0 = s * rows_pt
        pltpu.sync_copy(src3d.at[s], sidx)
        pltpu.sync_copy(dst3d.at[s], didx)

        def run(tab):
            def init(nr):
                pltpu.sync_copy(tab.at[pl.ds(r0, nr), :],
                                acc.at[pl.ds(r0, nr), :])
            _ranged(init, s, rows_pt, n)
            plsc.subcore_barrier()

            def body(j, carry):
                pltpu.async_copy(tab.at[sidx.at[j]], rows_v, sem).wait()
                pltpu.sync_copy(rows_v, acc.at[didx.at[j]], add=True)
                return carry
            lax.fori_loop(0, n_chunks, body, 0)

        pl.when(c == 0)(functools.partial(run, hs0))
        pl.when(c == 1)(functools.partial(run, hs1))
        plsc.subcore_barrier()

        def emit(nr):
            pltpu.sync_copy(acc.at[pl.ds(r0, nr), :],
                            out_hbm.at[c, pl.ds(r0, nr), :])
        _ranged(emit, s, rows_pt, n)

    return sc1_k


def _make_scatter2(n, np_rows, e_pad, ncls):
    n_chunks = e_pad // (NC * NS * CHUNK)
    rows_pt = np_rows // NS

    @functools.partial(
        pl.kernel,
        out_type=jax.ShapeDtypeStruct((NC, n, ncls), jnp.float32),
        mesh=_mesh(),
        scratch_types=[
            pltpu.VMEM((n_chunks, CHUNK), jnp.int32),
            pltpu.VMEM((n_chunks, CHUNK), jnp.int32),
            pltpu.VMEM((CHUNK, ncls), jnp.float32),
            pltpu.VMEM_SHARED((np_rows, ncls), jnp.float32),
            pltpu.SemaphoreType.DMA,
        ],
    )
    def sc2_k(src3d, dst3d, g2s, zeros_hbm, out_hbm,
              sidx, didx, rows_v, acc, sem):
        c = lax.axis_index("c")
        s = lax.axis_index("s")
        wid = s * NC + c
        r0 = s * rows_pt
        pltpu.sync_copy(src3d.at[wid], sidx)
        pltpu.sync_copy(dst3d.at[wid], didx)

        def init_from(tab, nr):
            pltpu.sync_copy(tab.at[pl.ds(r0, nr), :],
                            acc.at[pl.ds(r0, nr), :])
        pl.when(c == 0)(lambda: _ranged(
            functools.partial(init_from, g2s), s, rows_pt, n))
        pl.when(c == 1)(lambda: _ranged(
            functools.partial(init_from, zeros_hbm), s, rows_pt, n))
        plsc.subcore_barrier()

        def body(j, carry):
            pltpu.async_copy(g2s.at[sidx.at[j]], rows_v, sem).wait()
            pltpu.sync_copy(rows_v, acc.at[didx.at[j]], add=True)
            return carry
        lax.fori_loop(0, n_chunks, body, 0)
        plsc.subcore_barrier()

        def emit(nr):
            pltpu.sync_copy(acc.at[pl.ds(r0, nr), :],
                            out_hbm.at[c, pl.ds(r0, nr), :])
        _ranged(emit, s, rows_pt, n)

    return sc2_k


def _tc1_body(x_ref, w_ref, deg_ref, hs0_ref, hs1_ref, dinv_ref):
    deg = deg_ref[0, :, 0:1] + deg_ref[1, :, 0:1] + 1.0
    dinv = lax.rsqrt(deg)
    hs = jnp.dot(x_ref[:], w_ref[:], preferred_element_type=jnp.float32)
    hs = hs * dinv
    half = hs.shape[1] // 2
    hs0_ref[:] = hs[:, :half]
    hs1_ref[:] = hs[:, half:]
    dinv_ref[:] = dinv


def _tc2_body(agg_ref, dinv_ref, w2_ref, b1_ref, g_ref):
    dinv = dinv_ref[:]
    a = jnp.concatenate([agg_ref[0], agg_ref[1]], axis=1)
    out1 = jnp.maximum(a * dinv + b1_ref[:], 0.0)
    g = jnp.dot(out1, w2_ref[:], preferred_element_type=jnp.float32) * dinv
    ncls = g.shape[1]
    g_ref[:] = jnp.concatenate(
        [g, jnp.zeros((g.shape[0], 128 - ncls), jnp.float32)], axis=1)


def _tc3_body(p_ref, dinv_ref, b2_ref, out_ref):
    ncls = out_ref.shape[1]
    out_ref[:] = ((p_ref[0, :, :ncls] + p_ref[1, :, :ncls])
                  * dinv_ref[:] + b2_ref[:])


def kernel(x, edge_index, W1, b1, W2, b2):
    n, d = x.shape
    e = edge_index.shape[1]
    dh = W1.shape[1]
    ncls = W2.shape[1]
    half = dh // 2

    stride = NC * NS * CHUNK
    e_pad = -(-e // stride) * stride
    np_rows = -(-(n + 1) // (NS * 8)) * (NS * 8)

    src = edge_index[0]
    dst = edge_index[1]
    pe = e_pad - e
    src_p = jnp.concatenate([src, jnp.zeros((pe,), src.dtype)])
    dst_p = jnp.concatenate([dst, jnp.full((pe,), n, dst.dtype)])
    src32 = src_p.reshape(NC * NS, -1, CHUNK)
    dst32 = dst_p.reshape(NC * NS, -1, CHUNK)
    src16 = src_p.reshape(NS, -1, CHUNK)
    dst16 = dst_p.reshape(NS, -1, CHUNK)

    ones128 = jnp.ones((CHUNK, 128), jnp.float32)
    ncp = 128
    zeros128 = jnp.zeros((n, ncp), jnp.float32)

    degp = _make_deg_kernel(n, np_rows, e_pad)(dst32, ones128, zeros128)

    BR = 1000
    grid = (n // BR,)
    hs0, hs1, dinv = pl.pallas_call(
        _tc1_body,
        grid=grid,
        in_specs=[
            pl.BlockSpec((BR, d), lambda i: (i, 0)),
            pl.BlockSpec((d, dh), lambda i: (0, 0)),
            pl.BlockSpec((NC, BR, 128), lambda i: (0, i, 0)),
        ],
        out_specs=[
            pl.BlockSpec((BR, half), lambda i: (i, 0)),
            pl.BlockSpec((BR, half), lambda i: (i, 0)),
            pl.BlockSpec((BR, 1), lambda i: (i, 0)),
        ],
        out_shape=[
            jax.ShapeDtypeStruct((n, half), jnp.float32),
            jax.ShapeDtypeStruct((n, half), jnp.float32),
            jax.ShapeDtypeStruct((n, 1), jnp.float32),
        ],
    )(x, W1, degp)

    agg = _make_scatter1(n, np_rows, e_pad, half)(src16, dst16, hs0, hs1)

    g2s = pl.pallas_call(
        _tc2_body,
        grid=grid,
        in_specs=[
            pl.BlockSpec((NC, BR, half), lambda i: (0, i, 0)),
            pl.BlockSpec((BR, 1), lambda i: (i, 0)),
            pl.BlockSpec((dh, ncls), lambda i: (0, 0)),
            pl.BlockSpec((1, dh), lambda i: (0, 0)),
        ],
        out_specs=pl.BlockSpec((BR, ncp), lambda i: (i, 0)),
        out_shape=jax.ShapeDtypeStruct((n, ncp), jnp.float32),
    )(agg, dinv, W2, b1.reshape(1, dh))

    p2 = _make_scatter2(n, np_rows, e_pad, ncp)(src32, dst32, g2s, zeros128)

    out = pl.pallas_call(
        _tc3_body,
        grid=grid,
        in_specs=[
            pl.BlockSpec((NC, BR, ncp), lambda i: (0, i, 0)),
            pl.BlockSpec((BR, 1), lambda i: (i, 0)),
            pl.BlockSpec((1, ncls), lambda i: (0, 0)),
        ],
        out_specs=pl.BlockSpec((BR, ncls), lambda i: (i, 0)),
        out_shape=jax.ShapeDtypeStruct((n, ncls), jnp.float32),
    )(p2, dinv, b2.reshape(1, ncls))

    return out
